# baseline (speedup 1.0000x reference)
import functools

import jax
import jax.numpy as jnp
from jax import lax
from jax.experimental import pallas as pl
from jax.experimental.pallas import tpu as pltpu
from jax.experimental.pallas import tpu_sc as plsc

N_NODES = 10000
N_PAD = 10240
N_EDGES = 320000
ECH = 64
ROWS_PAD = 5120
E_PAD = ROWS_PAD * ECH
RT = ROWS_PAD // 32
IB = 32
NT = N_PAD // 16
PAD_NODE = N_NODES + 100
BR = 256
GRID = N_PAD // BR
D = 128
FH = 64
NBUF = 4

_MESH = plsc.VectorSubcoreMesh(core_axis_name="c", subcore_axis_name="s")


@functools.partial(
    pl.kernel,
    mesh=_MESH,
    out_type=jax.ShapeDtypeStruct((2, N_PAD, D), jnp.float32),
    scratch_types=[
        pltpu.VMEM((IB, ECH), jnp.int32),
        pltpu.VMEM((IB, ECH), jnp.int32),
        pltpu.VMEM((NBUF * ECH, D), jnp.float32),
        pltpu.VMEM_SHARED((N_PAD, D), jnp.float32),
    ] + [pltpu.SemaphoreType.DMA] * (2 * NBUF),
)
def _spmm(y_hbm, src_hbm, dst_hbm, zero_hbm, out_hbm, src_i, dst_i,
          rows_v, acc, *sems):
    rows = [rows_v.at[pl.ds(b * ECH, ECH)] for b in range(NBUF)]
    gsem = sems[:NBUF]
    ssem = sems[NBUF:]
    c = lax.axis_index("c")
    s = lax.axis_index("s")
    wid = c * 16 + s
    pltpu.sync_copy(zero_hbm.at[pl.ds(s * NT, NT)], acc.at[pl.ds(s * NT, NT)])
    plsc.subcore_barrier()

    for p in range(RT // IB):
        base = wid * RT + p * IB
        pltpu.sync_copy(src_hbm.at[pl.ds(base, IB)], src_i)
        pltpu.sync_copy(dst_hbm.at[pl.ds(base, IB)], dst_i)
        for b in range(NBUF):
            pltpu.async_copy(y_hbm.at[src_i.at[b]], rows[b], gsem[b])

        def body(k, carry):
            for b in range(NBUF):
                j = NBUF * k + b
                pltpu.make_async_copy(y_hbm.at[src_i.at[j]], rows[b],
                                      gsem[b]).wait()
                pltpu.sync_copy(rows[b], acc.at[dst_i.at[j]], add=True)
                nj = j + NBUF

                @pl.when(nj < IB)
                def _():
                    pltpu.async_copy(y_hbm.at[src_i.at[nj]], rows[b], gsem[b])
            return carry

        lax.fori_loop(0, IB // NBUF, body, 0)

    plsc.subcore_barrier()
    pltpu.sync_copy(acc.at[pl.ds(s * NT, NT)],
                    out_hbm.at[c, pl.ds(s * NT, NT)])


@functools.partial(
    pl.kernel,
    mesh=_MESH,
    out_type=jax.ShapeDtypeStruct((2, N_PAD, D), jnp.float32),
    scratch_types=[
        pltpu.VMEM((RT, ECH), jnp.int32),
        pltpu.VMEM((ECH, D), jnp.float32),
        pltpu.VMEM_SHARED((N_PAD, D), jnp.float32),
    ],
)
def _deg(ones_hbm, dst_hbm, zero_hbm, out_hbm, dst_v, ones_v, acc):
    c = lax.axis_index("c")
    s = lax.axis_index("s")
    wid = c * 16 + s
    pltpu.sync_copy(zero_hbm.at[pl.ds(s * NT, NT)], acc.at[pl.ds(s * NT, NT)])
    pltpu.sync_copy(dst_hbm.at[pl.ds(wid * RT, RT)], dst_v)
    pltpu.sync_copy(ones_hbm, ones_v)
    plsc.subcore_barrier()

    def body(j, carry):
        pltpu.sync_copy(ones_v, acc.at[dst_v.at[j]], add=True)
        return carry

    lax.fori_loop(0, RT, body, 0)
    plsc.subcore_barrier()
    pltpu.sync_copy(acc.at[pl.ds(s * NT, NT)],
                    out_hbm.at[c, pl.ds(s * NT, NT)])


def _dinv_block(degr):
    deg = degr[0, :, 0:1] + degr[1, :, 0:1] + 1.0
    return lax.rsqrt(deg)


def _tc_z_body(xr, wr, zo):
    zo[...] = jnp.dot(xr[...], wr[...], preferred_element_type=jnp.float32)


def _tc_z(xp, W):
    return pl.pallas_call(
        _tc_z_body,
        grid=(GRID,),
        in_specs=[
            pl.BlockSpec((BR, 128), lambda i: (i, 0)),
            pl.BlockSpec((128, 128), lambda i: (0, 0)),
        ],
        out_specs=pl.BlockSpec((BR, 128), lambda i: (i, 0)),
        out_shape=jax.ShapeDtypeStruct((N_PAD, 128), jnp.float32),
    )(xp, W)


def _tc_scale_body(zr, degr, yo):
    yo[...] = _dinv_block(degr) * zr[...]


def _tc_scale(z, degp):
    return pl.pallas_call(
        _tc_scale_body,
        grid=(GRID,),
        in_specs=[
            pl.BlockSpec((BR, 128), lambda i: (i, 0)),
            pl.BlockSpec((2, BR, D), lambda i: (0, i, 0)),
        ],
        out_specs=pl.BlockSpec((BR, 128), lambda i: (i, 0)),
        out_shape=jax.ShapeDtypeStruct((N_PAD, 128), jnp.float32),
    )(z, degp)


def _tc_mid_body(aggr, yr, degr, br, wr, yo):
    dinv = _dinv_block(degr)
    h = aggr[0] + aggr[1] + yr[...]
    h = jnp.maximum(dinv * h + br[...], 0.0)
    yo[...] = dinv * jnp.dot(h, wr[...], preferred_element_type=jnp.float32)


def _tc_mid(agg, y, degp, b, W):
    return pl.pallas_call(
        _tc_mid_body,
        grid=(GRID,),
        in_specs=[
            pl.BlockSpec((2, BR, 128), lambda i: (0, i, 0)),
            pl.BlockSpec((BR, 128), lambda i: (i, 0)),
            pl.BlockSpec((2, BR, D), lambda i: (0, i, 0)),
            pl.BlockSpec((1, 128), lambda i: (0, 0)),
            pl.BlockSpec((128, 128), lambda i: (0, 0)),
        ],
        out_specs=pl.BlockSpec((BR, 128), lambda i: (i, 0)),
        out_shape=jax.ShapeDtypeStruct((N_PAD, 128), jnp.float32),
    )(agg, y, degp, b, W)


def _tc_out_body(aggr, yr, degr, br, out):
    dinv = _dinv_block(degr)
    out[...] = dinv * (aggr[0] + aggr[1] + yr[...]) + br[...]


def _tc_out(agg, y, degp, b):
    return pl.pallas_call(
        _tc_out_body,
        grid=(GRID,),
        in_specs=[
            pl.BlockSpec((2, BR, 128), lambda i: (0, i, 0)),
            pl.BlockSpec((BR, 128), lambda i: (i, 0)),
            pl.BlockSpec((2, BR, D), lambda i: (0, i, 0)),
            pl.BlockSpec((1, 128), lambda i: (0, 0)),
        ],
        out_specs=pl.BlockSpec((BR, 128), lambda i: (i, 0)),
        out_shape=jax.ShapeDtypeStruct((N_PAD, 128), jnp.float32),
    )(agg, y, degp, b)


def kernel(x, edge_index, use_counts, counts, W1, b1, W2, b2, W3, b3):
    ei = edge_index.astype(jnp.int32)
    pad = jnp.full((E_PAD - N_EDGES,), PAD_NODE, jnp.int32)
    srcp = jnp.concatenate([ei[0], pad]).reshape(ROWS_PAD, ECH)
    dstp = jnp.concatenate([ei[1], pad]).reshape(ROWS_PAD, ECH)
    xp = jnp.pad(x, ((0, N_PAD - N_NODES), (0, 0)))
    W3p = jnp.pad(W3, ((0, 0), (0, 64)))
    b3p = jnp.pad(b3, (0, 64)).reshape(1, 128)

    zeros128 = jnp.zeros((N_PAD, D), jnp.float32)
    ones128 = jnp.ones((ECH, D), jnp.float32)

    z1 = _tc_z(xp, W1)
    degp = _deg(ones128, dstp, zeros128)
    y1 = _tc_scale(z1, degp)
    agg1 = _spmm(y1, srcp, dstp, zeros128)
    y2 = _tc_mid(agg1, y1, degp, b1.reshape(1, 128), W2)
    agg2 = _spmm(y2, srcp, dstp, zeros128)
    y3 = _tc_mid(agg2, y2, degp, b2.reshape(1, 128), W3p)
    agg3 = _spmm(y3, srcp, dstp, zeros128)
    out = _tc_out(agg3, y3, degp, b3p)
    return out[:N_NODES, :64]


# speedup vs baseline: 1.0971x; 1.0971x over previous
import functools

import jax
import jax.numpy as jnp
from jax import lax
from jax.experimental import pallas as pl
from jax.experimental.pallas import tpu as pltpu
from jax.experimental.pallas import tpu_sc as plsc

N_NODES = 10000
N_PAD = 10240
N_EDGES = 320000
ECH = 64
ROWS_PAD = 5120
E_PAD = ROWS_PAD * ECH
RT = ROWS_PAD // 32
IB = 32
NT = N_PAD // 16
PAD_NODE = N_NODES + 100
BR = 256
GRID = N_PAD // BR
D = 128
FH = 64
NBUF = 4

_MESH = plsc.VectorSubcoreMesh(core_axis_name="c", subcore_axis_name="s")


@functools.partial(
    pl.kernel,
    mesh=_MESH,
    out_type=jax.ShapeDtypeStruct((2, N_PAD, D), jnp.float32),
    scratch_types=[
        pltpu.VMEM((IB, ECH), jnp.int32),
        pltpu.VMEM((IB, ECH), jnp.int32),
        pltpu.VMEM((NBUF * ECH, D), jnp.float32),
        pltpu.VMEM_SHARED((N_PAD, D), jnp.float32),
    ] + [pltpu.SemaphoreType.DMA] * (2 * NBUF),
)
def _spmm(y_hbm, src_hbm, dst_hbm, zero_hbm, out_hbm, src_i, dst_i,
          rows_v, acc, *sems):
    rows = [rows_v.at[pl.ds(b * ECH, ECH)] for b in range(NBUF)]
    gsem = sems[:NBUF]
    ssem = sems[NBUF:]
    c = lax.axis_index("c")
    s = lax.axis_index("s")
    wid = c * 16 + s
    pltpu.sync_copy(zero_hbm.at[pl.ds(s * NT, NT)], acc.at[pl.ds(s * NT, NT)])
    plsc.subcore_barrier()

    for p in range(RT // IB):
        base = wid * RT + p * IB
        pltpu.sync_copy(src_hbm.at[pl.ds(base, IB)], src_i)
        pltpu.sync_copy(dst_hbm.at[pl.ds(base, IB)], dst_i)
        for b in range(NBUF):
            pltpu.async_copy(y_hbm.at[src_i.at[b]], rows[b], gsem[b])

        def body(k, carry):
            for b in range(NBUF):
                j = NBUF * k + b
                pltpu.make_async_copy(y_hbm.at[src_i.at[j]], rows[b],
                                      gsem[b]).wait()
                pltpu.sync_copy(rows[b], acc.at[dst_i.at[j]], add=True)
                nj = j + NBUF

                @pl.when(nj < IB)
                def _():
                    pltpu.async_copy(y_hbm.at[src_i.at[nj]], rows[b], gsem[b])
            return carry

        lax.fori_loop(0, IB // NBUF, body, 0)

    plsc.subcore_barrier()
    pltpu.sync_copy(acc.at[pl.ds(s * NT, NT)],
                    out_hbm.at[c, pl.ds(s * NT, NT)])


@functools.partial(
    pl.kernel,
    mesh=_MESH,
    out_type=jax.ShapeDtypeStruct((2, N_PAD, D), jnp.float32),
    scratch_types=[
        pltpu.VMEM((RT, ECH), jnp.int32),
        pltpu.VMEM((ECH, D), jnp.float32),
        pltpu.VMEM_SHARED((N_PAD, D), jnp.float32),
    ],
)
def _deg(ones_hbm, dst_hbm, zero_hbm, out_hbm, dst_v, ones_v, acc):
    c = lax.axis_index("c")
    s = lax.axis_index("s")
    wid = c * 16 + s
    pltpu.sync_copy(zero_hbm.at[pl.ds(s * NT, NT)], acc.at[pl.ds(s * NT, NT)])
    pltpu.sync_copy(dst_hbm.at[pl.ds(wid * RT, RT)], dst_v)
    pltpu.sync_copy(ones_hbm, ones_v)
    plsc.subcore_barrier()

    def body(j, carry):
        pltpu.sync_copy(ones_v, acc.at[dst_v.at[j]], add=True)
        return carry

    lax.fori_loop(0, RT, body, 0)
    plsc.subcore_barrier()
    pltpu.sync_copy(acc.at[pl.ds(s * NT, NT)],
                    out_hbm.at[c, pl.ds(s * NT, NT)])


def _dinv_block(degr):
    deg = degr[0, :, 0:1] + degr[1, :, 0:1] + 1.0
    return lax.rsqrt(deg)


def _tc_first_body(xr, degr, wr, yo):
    dinv = _dinv_block(degr)
    yo[...] = dinv * jnp.dot(xr[...], wr[...],
                             preferred_element_type=jnp.float32)


def _tc_first(xp, degp, W):
    return pl.pallas_call(
        _tc_first_body,
        grid=(GRID,),
        in_specs=[
            pl.BlockSpec((BR, 128), lambda i: (i, 0)),
            pl.BlockSpec((2, BR, D), lambda i: (0, i, 0)),
            pl.BlockSpec((128, 128), lambda i: (0, 0)),
        ],
        out_specs=pl.BlockSpec((BR, 128), lambda i: (i, 0)),
        out_shape=jax.ShapeDtypeStruct((N_PAD, 128), jnp.float32),
    )(xp, degp, W)


def _tc_mid_body(aggr, yr, degr, br, wr, yo):
    dinv = _dinv_block(degr)
    h = aggr[0] + aggr[1] + yr[...]
    h = jnp.maximum(dinv * h + br[...], 0.0)
    yo[...] = dinv * jnp.dot(h, wr[...], preferred_element_type=jnp.float32)


def _tc_mid(agg, y, degp, b, W):
    return pl.pallas_call(
        _tc_mid_body,
        grid=(GRID,),
        in_specs=[
            pl.BlockSpec((2, BR, 128), lambda i: (0, i, 0)),
            pl.BlockSpec((BR, 128), lambda i: (i, 0)),
            pl.BlockSpec((2, BR, D), lambda i: (0, i, 0)),
            pl.BlockSpec((1, 128), lambda i: (0, 0)),
            pl.BlockSpec((128, 128), lambda i: (0, 0)),
        ],
        out_specs=pl.BlockSpec((BR, 128), lambda i: (i, 0)),
        out_shape=jax.ShapeDtypeStruct((N_PAD, 128), jnp.float32),
    )(agg, y, degp, b, W)


def _tc_out_body(aggr, yr, degr, br, out):
    dinv = _dinv_block(degr)
    out[...] = dinv * (aggr[0] + aggr[1] + yr[...]) + br[...]


def _tc_out(agg, y, degp, b):
    return pl.pallas_call(
        _tc_out_body,
        grid=(GRID,),
        in_specs=[
            pl.BlockSpec((2, BR, 128), lambda i: (0, i, 0)),
            pl.BlockSpec((BR, 128), lambda i: (i, 0)),
            pl.BlockSpec((2, BR, D), lambda i: (0, i, 0)),
            pl.BlockSpec((1, 128), lambda i: (0, 0)),
        ],
        out_specs=pl.BlockSpec((BR, 128), lambda i: (i, 0)),
        out_shape=jax.ShapeDtypeStruct((N_PAD, 128), jnp.float32),
    )(agg, y, degp, b)


def kernel(x, edge_index, use_counts, counts, W1, b1, W2, b2, W3, b3):
    ei = edge_index.astype(jnp.int32)
    pad = jnp.full((E_PAD - N_EDGES,), PAD_NODE, jnp.int32)
    srcp = jnp.concatenate([ei[0], pad]).reshape(ROWS_PAD, ECH)
    dstp = jnp.concatenate([ei[1], pad]).reshape(ROWS_PAD, ECH)
    xp = jnp.pad(x, ((0, N_PAD - N_NODES), (0, 0)))
    W3p = jnp.pad(W3, ((0, 0), (0, 64)))
    b3p = jnp.pad(b3, (0, 64)).reshape(1, 128)

    zeros128 = jnp.zeros((N_PAD, D), jnp.float32)
    ones128 = jnp.ones((ECH, D), jnp.float32)

    degp = _deg(ones128, dstp, zeros128)
    y1 = _tc_first(xp, degp, W1)
    agg1 = _spmm(y1, srcp, dstp, zeros128)
    y2 = _tc_mid(agg1, y1, degp, b1.reshape(1, 128), W2)
    agg2 = _spmm(y2, srcp, dstp, zeros128)
    y3 = _tc_mid(agg2, y2, degp, b2.reshape(1, 128), W3p)
    agg3 = _spmm(y3, srcp, dstp, zeros128)
    out = _tc_out(agg3, y3, degp, b3p)
    return out[:N_NODES, :64]
